# baseline (speedup 1.0000x reference)
import jax
import jax.numpy as jnp
from jax.experimental import pallas as pl
from jax.experimental.pallas import tpu as pltpu

N_REGION = 100000
N_SLOT = 8
HIDDEN = 64
ROW = N_SLOT * HIDDEN
BR = 1000
NB = N_REGION // BR
S = 12
D = 8


def _body(rg_ref, mem_hbm, oemb_ref, w1_ref, w2_ref, sel_ref, selt_ref,
          out_hbm, bufs, win_v, sem_in, sem_out, rsem):
    def in_cp(b):
        s = b % S
        return pltpu.make_async_copy(
            mem_hbm.at[pl.ds(b * BR, BR)], bufs.at[s], sem_in.at[s])

    def out_cp(b):
        s = b % S
        return pltpu.make_async_copy(
            bufs.at[s], out_hbm.at[pl.ds(b * BR, BR)], sem_out.at[s])

    for b in range(D):
        in_cp(b).start()

    rg = rg_ref[0]
    j = rg % 8
    base = pl.multiple_of(rg - j, 8)
    win_cp = pltpu.make_async_copy(mem_hbm.at[pl.ds(base, 8)], win_v, rsem)
    win_cp.start()
    win_cp.wait()
    win = win_v[...]
    ids = jax.lax.broadcasted_iota(jnp.int32, (8, 1), 0)
    mask = ids == j
    row = jnp.sum(jnp.where(mask, win, 0.0), axis=0, keepdims=True)
    c0 = jax.lax.dot(oemb_ref[...] * w1_ref[...], sel_ref[...],
                     preferred_element_type=jnp.float32)
    d = jax.lax.dot(row * w2_ref[...], sel_ref[...],
                    preferred_element_type=jnp.float32)
    g = jax.nn.sigmoid(c0 + d)
    ge = jax.lax.dot(g, selt_ref[...],
                     preferred_element_type=jnp.float32)
    new_row = row * (1.0 - ge) + oemb_ref[...] * ge
    win_v[...] = jnp.where(mask, new_row, win)

    for b in range(NB):
        in_cp(b).wait()
        out_cp(b).start()
        c = b + D
        if c < NB:
            if c >= S:
                out_cp(c - S).wait()
            in_cp(c).start()
    for b in range(max(0, NB - S), NB):
        out_cp(b).wait()

    fin = pltpu.make_async_copy(win_v, out_hbm.at[pl.ds(base, 8)], rsem)
    fin.start()
    fin.wait()


def kernel(memory, o_emb, W_fg, o_rg):
    mem2d = memory.reshape(N_REGION, ROW)
    oemb512 = jnp.tile(o_emb, N_SLOT).reshape(1, ROW)
    w1_512 = jnp.tile(W_fg[0, :HIDDEN], N_SLOT).reshape(1, ROW)
    w2_512 = jnp.tile(W_fg[0, HIDDEN:], N_SLOT).reshape(1, ROW)
    sel = (jnp.arange(ROW, dtype=jnp.int32)[:, None] // HIDDEN
           == jnp.arange(N_SLOT, dtype=jnp.int32)[None, :]).astype(jnp.float32)
    rg = jnp.asarray(o_rg, jnp.int32).reshape((1,))

    out = pl.pallas_call(
        _body,
        in_specs=[
            pl.BlockSpec(memory_space=pltpu.MemorySpace.SMEM),
            pl.BlockSpec(memory_space=pltpu.MemorySpace.HBM),
            pl.BlockSpec(memory_space=pltpu.MemorySpace.VMEM),
            pl.BlockSpec(memory_space=pltpu.MemorySpace.VMEM),
            pl.BlockSpec(memory_space=pltpu.MemorySpace.VMEM),
            pl.BlockSpec(memory_space=pltpu.MemorySpace.VMEM),
            pl.BlockSpec(memory_space=pltpu.MemorySpace.VMEM),
        ],
        out_specs=pl.BlockSpec(memory_space=pltpu.MemorySpace.HBM),
        out_shape=jax.ShapeDtypeStruct((N_REGION, ROW), jnp.float32),
        scratch_shapes=[
            pltpu.VMEM((S, BR, ROW), jnp.float32),
            pltpu.VMEM((8, ROW), jnp.float32),
            pltpu.SemaphoreType.DMA((S,)),
            pltpu.SemaphoreType.DMA((S,)),
            pltpu.SemaphoreType.DMA,
        ],
    )(rg, mem2d, oemb512, w1_512, w2_512, sel, sel.T)
    return out.reshape(N_REGION, N_SLOT, HIDDEN)
